# Initial kernel scaffold; baseline (speedup 1.0000x reference)
import jax
import jax.numpy as jnp
from jax.experimental import pallas as pl


def kernel(x, adj, W):
    raise NotImplementedError("write your pallas kernel here")


# speedup vs baseline: 1.5821x; 1.5821x over previous
import jax
import jax.numpy as jnp
from jax.experimental import pallas as pl
from jax.experimental.pallas import tpu as pltpu

_N = 10000
_BM = 1000
_BK = 2048
_NK = (_N + _BK - 1) // _BK


def _sage_kernel(adj_ref, xk_ref, xi_ref, w1t_ref, w2t_ref, out_ref,
                 acc_ref, deg_ref):
    k = pl.program_id(1)

    @pl.when(k == 0)
    def _init():
        acc_ref[...] = jnp.zeros_like(acc_ref)
        deg_ref[...] = jnp.zeros_like(deg_ref)

    a = adj_ref[...]
    xk = xk_ref[...]

    @pl.when(k == _NK - 1)
    def _masked_tail():
        col = k * _BK + jax.lax.broadcasted_iota(jnp.int32, a.shape, 1)
        am = jnp.where(col < _N, a, 0.0)
        row = k * _BK + jax.lax.broadcasted_iota(jnp.int32, xk.shape, 0)
        xm = jnp.where(row < _N, xk, 0.0)
        acc_ref[...] += jnp.dot(am, xm, preferred_element_type=jnp.float32)
        deg_ref[...] += jnp.sum(am, axis=1, keepdims=True)

    @pl.when(k < _NK - 1)
    def _full():
        acc_ref[...] += jnp.dot(a, xk, preferred_element_type=jnp.float32)
        deg_ref[...] += jnp.sum(a, axis=1, keepdims=True)

    @pl.when(k == _NK - 1)
    def _epilogue():
        neigh = acc_ref[...] / (deg_ref[...] + 1.0)
        h = jnp.dot(xi_ref[...], w1t_ref[...],
                    preferred_element_type=jnp.float32)
        h += jnp.dot(neigh, w2t_ref[...],
                     preferred_element_type=jnp.float32)
        out_ref[...] = jnp.maximum(h, 0.0)


@jax.jit
def kernel(x, adj, W):
    nfeat = x.shape[1]
    nembed = W.shape[0]
    w1t = W[:, :nfeat].T
    w2t = W[:, nfeat:].T

    grid = (_N // _BM, _NK)
    return pl.pallas_call(
        _sage_kernel,
        grid=grid,
        in_specs=[
            pl.BlockSpec((_BM, _BK), lambda i, k: (i, k)),
            pl.BlockSpec((_BK, nfeat), lambda i, k: (k, 0)),
            pl.BlockSpec((_BM, nfeat), lambda i, k: (i, 0)),
            pl.BlockSpec((nfeat, nembed), lambda i, k: (0, 0)),
            pl.BlockSpec((nfeat, nembed), lambda i, k: (0, 0)),
        ],
        out_specs=pl.BlockSpec((_BM, nembed), lambda i, k: (i, 0)),
        out_shape=jax.ShapeDtypeStruct((_N, nembed), jnp.float32),
        scratch_shapes=[
            pltpu.VMEM((_BM, nfeat), jnp.float32),
            pltpu.VMEM((_BM, 1), jnp.float32),
        ],
        compiler_params=pltpu.CompilerParams(
            dimension_semantics=("parallel", "arbitrary"),
        ),
    )(adj, x, x, w1t, w2t)


# speedup vs baseline: 1.8880x; 1.1934x over previous
import jax
import jax.numpy as jnp
from jax.experimental import pallas as pl
from jax.experimental.pallas import tpu as pltpu

_N = 10000
_BM = 400


def _sage_kernel(adj_ref, x_ref, xi_ref, w1t_ref, w2t_ref, out_ref):
    a = adj_ref[...]
    s = jnp.dot(a, x_ref[...], preferred_element_type=jnp.float32)
    deg = jnp.sum(a, axis=1, keepdims=True)
    neigh = s / (deg + 1.0)
    h = jnp.dot(xi_ref[...], w1t_ref[...], preferred_element_type=jnp.float32)
    h += jnp.dot(neigh, w2t_ref[...], preferred_element_type=jnp.float32)
    out_ref[...] = jnp.maximum(h, 0.0)


@jax.jit
def kernel(x, adj, W):
    nfeat = x.shape[1]
    nembed = W.shape[0]
    w1t = W[:, :nfeat].T
    w2t = W[:, nfeat:].T

    return pl.pallas_call(
        _sage_kernel,
        grid=(_N // _BM,),
        in_specs=[
            pl.BlockSpec((_BM, _N), lambda i: (i, 0)),
            pl.BlockSpec((_N, nfeat), lambda i: (0, 0)),
            pl.BlockSpec((_BM, nfeat), lambda i: (i, 0)),
            pl.BlockSpec((nfeat, nembed), lambda i: (0, 0)),
            pl.BlockSpec((nfeat, nembed), lambda i: (0, 0)),
        ],
        out_specs=pl.BlockSpec((_BM, nembed), lambda i: (i, 0)),
        out_shape=jax.ShapeDtypeStruct((_N, nembed), jnp.float32),
        compiler_params=pltpu.CompilerParams(
            dimension_semantics=("arbitrary",),
        ),
    )(adj, x, x, w1t, w2t)
